# baseline (speedup 1.0000x reference)
import functools

import jax
import jax.numpy as jnp
from jax import lax
from jax.experimental import pallas as pl
from jax.experimental.pallas import tpu as pltpu
from jax.experimental.pallas import tpu_sc as plsc

NC = 2
NS = 16
NW = NC * NS
B = 16384
D = 64
V = 100000
ROWS_PER_W = B // NW
CHUNK = 128
NCHUNK = ROWS_PER_W // CHUNK
INV_B = 1.0 / B
PW = 16384
FW = 16384


def _tpose_pad_body(x_ref, out_ref):
    out_ref[:, 0:D] = x_ref[...].T


def _tpose_pair_body(x_ref, out_ref):
    w = x_ref.shape[1]
    out_ref[...] = x_ref[...].T.reshape(w // 2, 128)


def _row_major_pad(x_t, n_rows, w):
    return pl.pallas_call(
        _tpose_pad_body,
        grid=((n_rows + w - 1) // w,),
        in_specs=[pl.BlockSpec((D, w), lambda i: (0, i))],
        out_specs=pl.BlockSpec((w, 128), lambda i: (i, 0)),
        out_shape=jax.ShapeDtypeStruct((n_rows, 128), jnp.float32),
    )(x_t)


def _row_major_pair(x_t, n_rows, w):
    return pl.pallas_call(
        _tpose_pair_body,
        grid=(n_rows // w,),
        in_specs=[pl.BlockSpec((D, w), lambda i: (0, i))],
        out_specs=pl.BlockSpec((w // 2, 128), lambda i: (i, 0)),
        out_shape=jax.ShapeDtypeStruct((n_rows // 2, 128), jnp.float32),
    )(x_t)


def _sc_body(labels_hbm, feat_hbm, table_hbm, out_hbm,
             lab_v, feat_v0, feat_v1, rows_v0, rows_v1, acc_v,
             gsem0, gsem1, fsem0, fsem1):
    wid = lax.axis_index("s") * NC + lax.axis_index("c")
    base = wid * ROWS_PER_W
    fbase = wid * (ROWS_PER_W // 2)

    pltpu.sync_copy(labels_hbm.at[pl.ds(wid * NCHUNK, NCHUNK)], lab_v)

    feat_bufs = [feat_v0, feat_v1]
    rows_bufs = [rows_v0, rows_v1]
    gsems = [gsem0, gsem1]
    fsems = [fsem0, fsem1]

    def start(j):
        g = pltpu.async_copy(table_hbm.at[lab_v.at[j]], rows_bufs[j % 2],
                             gsems[j % 2])
        f = pltpu.async_copy(
            feat_hbm.at[pl.ds(base + j * CHUNK, CHUNK)],
            feat_bufs[j % 2], fsems[j % 2])
        return g, f

    accs = [jnp.zeros((16,), jnp.float32) for _ in range(8)]
    pend = start(0)
    for j in range(NCHUNK):
        nxt = start(j + 1) if j + 1 < NCHUNK else None
        pend[0].wait()
        pend[1].wait()
        rb = rows_bufs[j % 2]
        fb = feat_bufs[j % 2]

        def pair_body(t, carry):
            outs = list(carry)
            for half in range(2):
                for c in range(4):
                    fv = fb[2 * t + half, pl.ds(c * 16, 16)]
                    pv = rb[2 * t + half, pl.ds(c * 16, 16)]
                    d = jnp.abs(fv - pv)
                    m = jnp.minimum(d, 1.0)
                    k = half * 4 + c
                    outs[k] = outs[k] + m * (d - 0.5 * m)
            return tuple(outs)

        accs = list(lax.fori_loop(0, CHUNK // 2, pair_body, tuple(accs)))
        pend = nxt

    for k in range(8):
        acc_v[pl.ds(k * 16, 16)] = accs[k] * INV_B
    pltpu.sync_copy(acc_v, out_hbm.at[wid])


@jax.jit
def kernel(features, proxy, labels):
    labels2d = labels.astype(jnp.int32).reshape(B // CHUNK, CHUNK)
    table128 = _row_major_pad(proxy.T, V, PW)
    feat128 = _row_major_pad(features.T, B, FW)
    run = pl.kernel(
        _sc_body,
        out_type=jax.ShapeDtypeStruct((NW, 128), jnp.float32),
        mesh=plsc.VectorSubcoreMesh(core_axis_name="c", subcore_axis_name="s"),
        compiler_params=pltpu.CompilerParams(
            use_tc_tiling_on_sc=True, needs_layout_passes=False),
        scratch_types=[
            pltpu.VMEM((NCHUNK, CHUNK), jnp.int32),
            pltpu.VMEM((CHUNK, 128), jnp.float32),
            pltpu.VMEM((CHUNK, 128), jnp.float32),
            pltpu.VMEM((CHUNK, 128), jnp.float32),
            pltpu.VMEM((CHUNK, 128), jnp.float32),
            pltpu.VMEM((128,), jnp.float32),
            pltpu.SemaphoreType.DMA,
            pltpu.SemaphoreType.DMA,
            pltpu.SemaphoreType.DMA,
            pltpu.SemaphoreType.DMA,
        ],
    )
    partials = run(labels2d, feat128, table128)
    return jnp.sum(partials)


# speedup vs baseline: 1.0195x; 1.0195x over previous
import functools

import jax
import jax.numpy as jnp
from jax import lax
from jax.experimental import pallas as pl
from jax.experimental.pallas import tpu as pltpu
from jax.experimental.pallas import tpu_sc as plsc

NC = 2
NS = 16
NW = NC * NS
B = 16384
D = 64
V = 100000
ROWS_PER_W = B // NW
CHUNK = 128
NCHUNK = ROWS_PER_W // CHUNK
INV_B = 1.0 / B
PW = 16384
FW = 16384


def _tpose_pad_body(x_ref, out_ref):
    out_ref[:, 0:D] = x_ref[...].T


def _tpose_pair_body(x_ref, out_ref):
    w = x_ref.shape[1]
    out_ref[...] = x_ref[...].T.reshape(w // 2, 128)


def _row_major_pad(x_t, n_rows, w):
    return pl.pallas_call(
        _tpose_pad_body,
        grid=((n_rows + w - 1) // w,),
        in_specs=[pl.BlockSpec((D, w), lambda i: (0, i))],
        out_specs=pl.BlockSpec((w, 128), lambda i: (i, 0)),
        out_shape=jax.ShapeDtypeStruct((n_rows, 128), jnp.float32),
    )(x_t)


def _row_major_pair(x_t, n_rows, w):
    return pl.pallas_call(
        _tpose_pair_body,
        grid=(n_rows // w,),
        in_specs=[pl.BlockSpec((D, w), lambda i: (0, i))],
        out_specs=pl.BlockSpec((w // 2, 128), lambda i: (i, 0)),
        out_shape=jax.ShapeDtypeStruct((n_rows // 2, 128), jnp.float32),
    )(x_t)


def _sc_body(labels_hbm, feat_hbm, table_hbm, out_hbm,
             lab_v, feat_v, rows_v0, rows_v1, acc_v,
             gsem0, gsem1, fsem):
    wid = lax.axis_index("s") * NC + lax.axis_index("c")
    base = wid * ROWS_PER_W

    f = pltpu.async_copy(feat_hbm.at[pl.ds(base, ROWS_PER_W)], feat_v, fsem)
    pltpu.sync_copy(labels_hbm.at[pl.ds(wid * NCHUNK, NCHUNK)], lab_v)

    rows_bufs = [rows_v0, rows_v1]
    gsems = [gsem0, gsem1]

    def start(j):
        return pltpu.async_copy(table_hbm.at[lab_v.at[j]], rows_bufs[j % 2],
                                gsems[j % 2])

    accs = [jnp.zeros((16,), jnp.float32) for _ in range(8)]
    pend = start(0)
    f.wait()
    for j in range(NCHUNK):
        nxt = start(j + 1) if j + 1 < NCHUNK else None
        pend.wait()
        rb = rows_bufs[j % 2]
        foff = j * CHUNK

        def pair_body(t, carry):
            outs = list(carry)
            for half in range(2):
                for c in range(4):
                    fv = feat_v[foff + 2 * t + half, pl.ds(c * 16, 16)]
                    pv = rb[2 * t + half, pl.ds(c * 16, 16)]
                    d = jnp.abs(fv - pv)
                    m = jnp.minimum(d, 1.0)
                    k = half * 4 + c
                    outs[k] = outs[k] + m * (d - 0.5 * m)
            return tuple(outs)

        accs = list(lax.fori_loop(0, CHUNK // 2, pair_body, tuple(accs)))
        pend = nxt

    for k in range(8):
        acc_v[pl.ds(k * 16, 16)] = accs[k] * INV_B
    pltpu.sync_copy(acc_v, out_hbm.at[wid])


@jax.jit
def kernel(features, proxy, labels):
    labels2d = labels.astype(jnp.int32).reshape(B // CHUNK, CHUNK)
    table128 = _row_major_pad(proxy.T, V, PW)
    feat128 = _row_major_pad(features.T, B, FW)
    run = pl.kernel(
        _sc_body,
        out_type=jax.ShapeDtypeStruct((NW, 128), jnp.float32),
        mesh=plsc.VectorSubcoreMesh(core_axis_name="c", subcore_axis_name="s"),
        compiler_params=pltpu.CompilerParams(
            use_tc_tiling_on_sc=True, needs_layout_passes=False),
        scratch_types=[
            pltpu.VMEM((NCHUNK, CHUNK), jnp.int32),
            pltpu.VMEM((ROWS_PER_W, 128), jnp.float32),
            pltpu.VMEM((CHUNK, 128), jnp.float32),
            pltpu.VMEM((CHUNK, 128), jnp.float32),
            pltpu.VMEM((128,), jnp.float32),
            pltpu.SemaphoreType.DMA,
            pltpu.SemaphoreType.DMA,
            pltpu.SemaphoreType.DMA,
        ],
    )
    partials = run(labels2d, feat128, table128)
    return jnp.sum(partials)


# speedup vs baseline: 1.0208x; 1.0012x over previous
import jax
import jax.numpy as jnp
from jax import lax
from jax.experimental import pallas as pl
from jax.experimental.pallas import tpu as pltpu
from jax.experimental.pallas import tpu_sc as plsc

NC = 2
NS = 16
NW = NC * NS
B = 16384
D = 64
V = 100000
ROWS_PER_W = B // NW
CHUNK = 128
NCHUNK = ROWS_PER_W // CHUNK
INV_B = 1.0 / B
PW = 16384
FW = 16384


def _tpose_pad_body(x_ref, out_ref):
    out_ref[:, 0:D] = x_ref[...].T


def _row_major_pad(x_t, n_rows, w):
    return pl.pallas_call(
        _tpose_pad_body,
        grid=((n_rows + w - 1) // w,),
        in_specs=[pl.BlockSpec((D, w), lambda i: (0, i))],
        out_specs=pl.BlockSpec((w, 128), lambda i: (i, 0)),
        out_shape=jax.ShapeDtypeStruct((n_rows, 128), jnp.float32),
    )(x_t)


def _sc_body(labels_hbm, feat_hbm, table_hbm, out_hbm,
             lab_v, feat_v, rows_v0, rows_v1, acc_v,
             gsem0, gsem1, fsem):
    wid = lax.axis_index("s") * NC + lax.axis_index("c")
    base = wid * ROWS_PER_W

    f = pltpu.async_copy(feat_hbm.at[pl.ds(base, ROWS_PER_W)], feat_v, fsem)
    pltpu.sync_copy(labels_hbm.at[pl.ds(wid * NCHUNK, NCHUNK)], lab_v)

    rows_bufs = [rows_v0, rows_v1]
    gsems = [gsem0, gsem1]

    def start(j):
        return pltpu.async_copy(table_hbm.at[lab_v.at[j]], rows_bufs[j % 2],
                                gsems[j % 2])

    accs = [jnp.zeros((16,), jnp.float32) for _ in range(8)]
    pend = start(0)
    f.wait()
    for j in range(NCHUNK):
        nxt = start(j + 1) if j + 1 < NCHUNK else None
        pend.wait()
        rb = rows_bufs[j % 2]
        foff = j * CHUNK

        def pair_body(t, carry):
            outs = list(carry)
            for half in range(2):
                for c in range(4):
                    fv = feat_v[foff + 2 * t + half, pl.ds(c * 16, 16)]
                    pv = rb[2 * t + half, pl.ds(c * 16, 16)]
                    d = jnp.abs(fv - pv)
                    m = jnp.minimum(d, 1.0)
                    k = half * 4 + c
                    outs[k] = outs[k] + m * (d - 0.5 * m)
            return tuple(outs)

        accs = list(lax.fori_loop(0, CHUNK // 2, pair_body, tuple(accs)))
        pend = nxt

    for k in range(8):
        acc_v[pl.ds(k * 16, 16)] = accs[k] * INV_B
    pltpu.sync_copy(acc_v, out_hbm.at[wid])


@jax.jit
def kernel(features, proxy, labels):
    labels2d = labels.astype(jnp.int32).reshape(B // CHUNK, CHUNK)
    table128 = _row_major_pad(proxy.T, V, PW)
    feat128 = _row_major_pad(features.T, B, FW)
    run = pl.kernel(
        _sc_body,
        out_type=jax.ShapeDtypeStruct((NW, 128), jnp.float32),
        mesh=plsc.VectorSubcoreMesh(core_axis_name="c", subcore_axis_name="s"),
        compiler_params=pltpu.CompilerParams(
            use_tc_tiling_on_sc=True, needs_layout_passes=False),
        scratch_types=[
            pltpu.VMEM((NCHUNK, CHUNK), jnp.int32),
            pltpu.VMEM((ROWS_PER_W, 128), jnp.float32),
            pltpu.VMEM((CHUNK, 128), jnp.float32),
            pltpu.VMEM((CHUNK, 128), jnp.float32),
            pltpu.VMEM((128,), jnp.float32),
            pltpu.SemaphoreType.DMA,
            pltpu.SemaphoreType.DMA,
            pltpu.SemaphoreType.DMA,
        ],
    )
    partials = run(labels2d, feat128, table128)
    return jnp.sum(partials)
